# Initial kernel scaffold; baseline (speedup 1.0000x reference)
import jax
import jax.numpy as jnp
from jax.experimental import pallas as pl


def kernel(x, edge_index, W1, b1, W_lin1, b_lin1, W_lin2, b_lin2):
    raise NotImplementedError("write your pallas kernel here")


# speedup vs baseline: 204.1721x; 204.1721x over previous
import functools

import jax
import jax.numpy as jnp
from jax import lax
from jax.experimental import pallas as pl
from jax.experimental.pallas import tpu as pltpu
from jax.experimental.pallas import tpu_sc as plsc

N = 100000
E = 6400000
NC = 2
NS = 16
NPAD = 100352
RN = NPAD // NS
PADC = NPAD - N

EPT1 = E // NS
C1 = 25000
NCH1 = EPT1 // C1

EPT2 = E // (NC * NS)
C2 = 25000
NCH2 = EPT2 // C2

F1 = 60
F2 = 30
F3 = 10


def _rsqrt_newton(d):
    i = lax.bitcast_convert_type(d, jnp.int32)
    i = jnp.int32(0x5F3759DF) - lax.shift_right_logical(i, 1)
    y = lax.bitcast_convert_type(i, jnp.float32)
    for _ in range(3):
        y = y * (jnp.float32(1.5) - jnp.float32(0.5) * d * y * y)
    return y


def _k1_body(src_hbm, dst_hbm, x_hbm, zeros_hbm, ones_hbm,
             h_out, indeg_out,
             deg_sh, idx_buf, ones_buf, deg_buf, x_buf, h_buf):
    c = lax.axis_index("c")
    s = lax.axis_index("s")
    sl = pl.ds(s * RN, RN)
    pltpu.sync_copy(zeros_hbm.at[sl], deg_sh.at[sl])
    pltpu.sync_copy(ones_hbm, ones_buf)
    plsc.subcore_barrier()

    def histo(idx_hbm):
        base = s * EPT1

        def body(k, carry):
            off = base + k * C1
            pltpu.sync_copy(idx_hbm.at[pl.ds(off, C1)], idx_buf)
            pltpu.sync_copy(ones_buf, deg_sh.at[idx_buf], add=True)
            return carry

        lax.fori_loop(0, NCH1, body, 0, unroll=False)

    @pl.when(c == 0)
    def _():
        histo(src_hbm)

    @pl.when(c == 1)
    def _():
        histo(dst_hbm)

    plsc.subcore_barrier()

    @pl.when(c == 0)
    def _():
        pltpu.sync_copy(deg_sh.at[sl], deg_buf)
        pltpu.sync_copy(x_hbm.at[sl], x_buf)

        def body(i, carry):
            v = pl.ds(i * 16, 16)
            d = jnp.maximum(deg_buf[v], jnp.float32(1.0))
            h_buf[v] = x_buf[v] * _rsqrt_newton(d)
            return carry

        lax.fori_loop(0, RN // 16, body, 0, unroll=False)
        pltpu.sync_copy(h_buf, h_out.at[sl])

    @pl.when(c == 1)
    def _():
        pltpu.sync_copy(deg_sh.at[sl], indeg_out.at[sl])


def _k2_body(src_hbm, dst_hbm, h_hbm, zeros_hbm,
             agg_out,
             h_sh, agg_sh, sidx, didx, msg):
    c = lax.axis_index("c")
    s = lax.axis_index("s")
    sl = pl.ds(s * RN, RN)
    pltpu.sync_copy(zeros_hbm.at[sl], agg_sh.at[sl])
    pltpu.sync_copy(h_hbm.at[sl], h_sh.at[sl])
    plsc.subcore_barrier()

    base = (c * NS + s) * EPT2

    def body(k, carry):
        off = base + k * C2
        pltpu.sync_copy(src_hbm.at[pl.ds(off, C2)], sidx)
        pltpu.sync_copy(dst_hbm.at[pl.ds(off, C2)], didx)
        pltpu.sync_copy(h_sh.at[sidx], msg)
        pltpu.sync_copy(msg, agg_sh.at[didx], add=True)
        return carry

    lax.fori_loop(0, NCH2, body, 0, unroll=False)
    plsc.subcore_barrier()
    pltpu.sync_copy(agg_sh.at[sl], agg_out.at[c, sl])


def _k3_body(aggp_ref, indeg_ref, w1_ref, b1_ref, wl1_ref, bl1_ref,
             wl2_ref, bl2_ref, out_ref, a_ref, hg_ref, h1_ref):
    a = (aggp_ref[0] + aggp_ref[1]) * lax.rsqrt(
        jnp.maximum(indeg_ref[...], jnp.float32(1.0)))
    a_ref[...] = a
    inv_n = jnp.float32(1.0 / N)
    for j in range(F1):
        w = w1_ref[j]
        b = b1_ref[j]
        colsum = jnp.sum(jnp.maximum(a_ref[...] * w + b, 0.0))
        colsum = colsum - PADC * jnp.maximum(b, 0.0)
        hg_ref[j] = jnp.maximum(colsum * inv_n, 0.0)

    def l1_body(k, carry):
        def inner(j, acc):
            return acc + hg_ref[j] * wl1_ref[k * F1 + j]

        acc = lax.fori_loop(0, F1, inner, bl1_ref[k])
        h1_ref[k] = jnp.maximum(acc, 0.0)
        return carry

    lax.fori_loop(0, F2, l1_body, 0)

    def l2_body(m, carry):
        def inner(k, acc):
            return acc + h1_ref[k] * wl2_ref[m * F2 + k]

        acc = lax.fori_loop(0, F2, inner, bl2_ref[m])
        out_ref[m] = jnp.maximum(acc, 0.0)
        return carry

    lax.fori_loop(0, F3, l2_body, 0)


def kernel(x, edge_index, W1, b1, W_lin1, b_lin1, W_lin2, b_lin2):
    src = edge_index[0].astype(jnp.int32)
    dst = edge_index[1].astype(jnp.int32)
    xp = jnp.pad(x[:, 0].astype(jnp.float32), (0, PADC))
    zeros = jnp.zeros((NPAD,), jnp.float32)
    ones = jnp.ones((C1,), jnp.float32)

    mesh = plsc.VectorSubcoreMesh(
        core_axis_name="c", subcore_axis_name="s",
        num_cores=NC, num_subcores=NS)

    h, indeg = pl.kernel(
        _k1_body,
        out_type=(
            jax.ShapeDtypeStruct((NPAD,), jnp.float32),
            jax.ShapeDtypeStruct((NPAD,), jnp.float32),
        ),
        mesh=mesh,
        scratch_types=[
            pltpu.VMEM_SHARED((NPAD,), jnp.float32),
            pltpu.VMEM((C1,), jnp.int32),
            pltpu.VMEM((C1,), jnp.float32),
            pltpu.VMEM((RN,), jnp.float32),
            pltpu.VMEM((RN,), jnp.float32),
            pltpu.VMEM((RN,), jnp.float32),
        ],
        name="gcn_degrees",
    )(src, dst, xp, zeros, ones)

    aggp = pl.kernel(
        _k2_body,
        out_type=jax.ShapeDtypeStruct((NC, NPAD), jnp.float32),
        mesh=mesh,
        scratch_types=[
            pltpu.VMEM_SHARED((NPAD,), jnp.float32),
            pltpu.VMEM_SHARED((NPAD,), jnp.float32),
            pltpu.VMEM((C2,), jnp.int32),
            pltpu.VMEM((C2,), jnp.int32),
            pltpu.VMEM((C2,), jnp.float32),
        ],
        name="gcn_messages",
    )(src, dst, h, zeros)

    out = pl.pallas_call(
        _k3_body,
        out_shape=jax.ShapeDtypeStruct((F3,), jnp.float32),
        in_specs=[
            pl.BlockSpec(memory_space=pltpu.VMEM),
            pl.BlockSpec(memory_space=pltpu.VMEM),
            pl.BlockSpec(memory_space=pltpu.SMEM),
            pl.BlockSpec(memory_space=pltpu.SMEM),
            pl.BlockSpec(memory_space=pltpu.SMEM),
            pl.BlockSpec(memory_space=pltpu.SMEM),
            pl.BlockSpec(memory_space=pltpu.SMEM),
            pl.BlockSpec(memory_space=pltpu.SMEM),
        ],
        out_specs=pl.BlockSpec(memory_space=pltpu.SMEM),
        scratch_shapes=[
            pltpu.VMEM((NPAD // 128, 128), jnp.float32),
            pltpu.SMEM((F1,), jnp.float32),
            pltpu.SMEM((F2,), jnp.float32),
        ],
        name="gcn_head",
    )(
        aggp.reshape(NC, NPAD // 128, 128),
        indeg.reshape(NPAD // 128, 128),
        W1.reshape(F1).astype(jnp.float32),
        b1.astype(jnp.float32),
        W_lin1.reshape(F2 * F1).astype(jnp.float32),
        b_lin1.astype(jnp.float32),
        W_lin2.reshape(F3 * F2).astype(jnp.float32),
        b_lin2.astype(jnp.float32),
    )
    return out.reshape(1, F3)
